# baseline (speedup 1.0000x reference)
import functools

import jax
import jax.numpy as jnp
from jax import lax
from jax.experimental import pallas as pl
from jax.experimental.pallas import tpu as pltpu
from jax.experimental.pallas import tpu_sc as plsc

N_NODES = 10000
N_EDGES = 320000
C = 64

NC = 2
NS = 16
NW = NC * NS
CH = 80
EPT = 10240
NCHUNK = EPT // CH
E_PAD = NW * EPT
NROWS = 10112
RPS = NROWS // NS


NBUF = 4
CPH = 16
NGRP = CPH // NBUF
NH0 = 3
NH1 = 13
EPT0 = NH0 * CPH * CH
EPT1 = NH1 * CPH * CH
E0 = NS * EPT0
E1 = NS * EPT1


def _sc_agg(h, src_t, dst_t, zrows):
  mesh = plsc.VectorSubcoreMesh(core_axis_name="c", subcore_axis_name="s")

  @functools.partial(
      pl.kernel,
      out_type=jax.ShapeDtypeStruct((NC, NROWS, 128), jnp.float32),
      mesh=mesh,
      scratch_types=[
          pltpu.VMEM((CPH, CH), jnp.int32),
          pltpu.VMEM((CPH, CH), jnp.int32),
          pltpu.VMEM((NBUF, CH, 128), jnp.float32),
          pltpu.VMEM_SHARED((NROWS, 128), jnp.float32),
      ] + [pltpu.SemaphoreType.DMA] * (2 * NBUF),
  )
  def k(h_hbm, src_hbm, dst_hbm, z_hbm, out_hbm, src_v, dst_v, rows, acc,
        *sems):
    gsem = sems[:NBUF]
    ssem = sems[NBUF:]
    cid = lax.axis_index("c")
    sid = lax.axis_index("s")
    pltpu.sync_copy(z_hbm.at[pl.ds(sid * RPS, RPS)],
                    acc.at[pl.ds(sid * RPS, RPS)])
    plsc.subcore_barrier()

    def round_body(half, carry):
      pltpu.sync_copy(src_hbm.at[cid, sid, pl.ds(half * CPH, CPH)], src_v)
      pltpu.sync_copy(dst_hbm.at[cid, sid, pl.ds(half * CPH, CPH)], dst_v)

      for b in range(NBUF):
        pltpu.async_copy(h_hbm.at[src_v.at[b]], rows.at[b], gsem[b])

      def body(m, carry):
        j0 = m * NBUF
        for b in range(NBUF):
          pltpu.make_async_copy(h_hbm.at[src_v.at[j0 + b]], rows.at[b],
                                gsem[b]).wait()
          pltpu.async_copy(rows.at[b], acc.at[dst_v.at[j0 + b]], ssem[b],
                           add=True)

        @pl.when(m < NGRP - 1)
        def _():
          for b in range(NBUF):
            pltpu.make_async_copy(rows.at[b], acc.at[dst_v.at[j0 + b]],
                                  ssem[b]).wait()
            pltpu.async_copy(h_hbm.at[src_v.at[j0 + NBUF + b]], rows.at[b],
                             gsem[b])

        return carry

      lax.fori_loop(0, NGRP, body, 0)
      for b in range(NBUF):
        pltpu.make_async_copy(rows.at[b], acc.at[dst_v.at[b]], ssem[b]).wait()
      return carry

    @pl.when(cid == 0)
    def _():
      lax.fori_loop(0, NH0, round_body, 0)

    @pl.when(cid == 1)
    def _():
      lax.fori_loop(0, NH1, round_body, 0)

    plsc.subcore_barrier()
    pltpu.sync_copy(acc.at[pl.ds(sid * RPS, RPS)],
                    out_hbm.at[cid, pl.ds(sid * RPS, RPS)])

  return k(h, src_t, dst_t, zrows)


def _sc_degree(dst_t, ones128, zrows):
  mesh = plsc.VectorSubcoreMesh(core_axis_name="c", subcore_axis_name="s")

  @functools.partial(
      pl.kernel,
      out_type=jax.ShapeDtypeStruct((NC, NROWS, 128), jnp.float32),
      mesh=mesh,
      scratch_types=[
          pltpu.VMEM((NCHUNK, CH), jnp.int32),
          pltpu.VMEM((CH, 128), jnp.float32),
          pltpu.VMEM_SHARED((NROWS, 128), jnp.float32),
          pltpu.SemaphoreType.DMA,
      ],
  )
  def k(dst_hbm, ones_hbm, z_hbm, out_hbm, dst_v, ones_v, acc, sem):
    cid = lax.axis_index("c")
    sid = lax.axis_index("s")
    wid = cid * NS + sid
    pltpu.sync_copy(z_hbm.at[pl.ds(sid * RPS, RPS)],
                    acc.at[pl.ds(sid * RPS, RPS)])
    pltpu.sync_copy(dst_hbm.at[wid], dst_v)
    pltpu.sync_copy(ones_hbm, ones_v)
    plsc.subcore_barrier()

    def fire(j, carry):
      pltpu.async_copy(ones_v, acc.at[dst_v.at[j]], sem, add=True)
      return carry

    lax.fori_loop(0, NCHUNK, fire, 0)

    def drain(j, carry):
      pltpu.make_async_copy(ones_v, acc.at[dst_v.at[0]], sem).wait()
      return carry

    lax.fori_loop(0, NCHUNK, drain, 0)
    plsc.subcore_barrier()
    pltpu.sync_copy(acc.at[pl.ds(sid * RPS, RPS)],
                    out_hbm.at[cid, pl.ds(sid * RPS, RPS)])

  return k(dst_t, ones128, zrows)


def _bn_cols(h, g, b):
  m = jnp.mean(h, axis=0, keepdims=True)
  v = jnp.mean((h - m) * (h - m), axis=0, keepdims=True)
  return (h - m) * lax.rsqrt(v + 1e-5) * g + b


def _stem_body(x_ref, w_ref, g_ref, b_ref, o_ref):
  h = jnp.dot(x_ref[...], w_ref[...], preferred_element_type=jnp.float32)
  o_ref[...] = _bn_cols(h, g_ref[...], b_ref[...])


def _tc_stem(x, w, g, b):
  return pl.pallas_call(
      _stem_body,
      out_shape=jax.ShapeDtypeStruct((N_NODES, w.shape[1]), jnp.float32),
  )(x, w, g, b)


def _pre_body(s0_ref, s1_ref, w0_ref, w1_ref, g_ref, b_ref, o_ref):
  h0 = jnp.dot(s0_ref[...], w0_ref[...], preferred_element_type=jnp.float32)
  h1 = jnp.dot(s1_ref[...], w1_ref[...], preferred_element_type=jnp.float32)
  h = jnp.concatenate([h0, h1], axis=1)
  o_ref[...] = jax.nn.relu(_bn_cols(h, g_ref[...], b_ref[...]))


def _tc_pre(s0, s1, w0, w1, g, b):
  return pl.pallas_call(
      _pre_body,
      out_shape=jax.ShapeDtypeStruct((N_NODES, 2 * C), jnp.float32),
  )(s0, s1, w0, w1, g, b)


def _rec_body(d_ref, o_ref):
  deg = d_ref[0, :N_NODES, 0:1] + d_ref[1, :N_NODES, 0:1]
  o_ref[...] = 1.0 / jnp.maximum(deg, 1.0)


def _tc_rec(degp):
  return pl.pallas_call(
      _rec_body,
      out_shape=jax.ShapeDtypeStruct((N_NODES, 1), jnp.float32),
  )(degp)


def _post_body(p_ref, r_ref, wa_ref, wb_ref, ska_ref, skb_ref, o_ref):
  agg = p_ref[0, :N_NODES, :] + p_ref[1, :N_NODES, :]
  rec = r_ref[...]
  a = jnp.dot(agg[:, :C], wa_ref[...], preferred_element_type=jnp.float32)
  bb = jnp.dot(agg[:, C:], wb_ref[...], preferred_element_type=jnp.float32)
  a = jax.nn.relu(a * rec) + ska_ref[...]
  bb = jax.nn.relu(bb * rec) + skb_ref[...]
  o_ref[...] = jnp.concatenate([a, bb], axis=1)


def _tc_post(p, rec, wa, wb, ska, skb):
  return pl.pallas_call(
      _post_body,
      out_shape=jax.ShapeDtypeStruct((N_NODES, 2 * C), jnp.float32),
  )(p, rec, wa, wb, ska, skb)


def _cls_body(s_ref, w0_ref, w1_ref, b_ref, o_ref):
  s = s_ref[...]
  mean = jnp.mean(s, axis=1, keepdims=True)
  o_ref[...] = (jnp.dot(s, w1_ref[...], preferred_element_type=jnp.float32)
                + mean * w0_ref[...] + b_ref[...])


def _tc_cls(s, w0, w1, b):
  return pl.pallas_call(
      _cls_body,
      out_shape=jax.ShapeDtypeStruct((N_NODES, w1.shape[1]), jnp.float32),
  )(s, w0, w1, b)


def kernel(x, edge_index, training, W_stem, g_stem, b_stem, W_pre, g_pre,
           b_pre, Wp0_0, Wp1_0, Wp0_1, Wp1_1, pre_g, pre_b, opW, W_cls,
           b_cls):
  src = edge_index[0]
  dst = edge_index[1]
  pad = E_PAD - N_EDGES
  dst_t = jnp.concatenate([dst, jnp.full((pad,), N_NODES, jnp.int32)]
                          ).reshape(NW, NCHUNK, CH)
  pad1 = E0 + E1 - N_EDGES
  padw = (NH1 - NH0) * CPH
  src0 = jnp.pad(src[:E0].reshape(NS, NH0 * CPH, CH),
                 ((0, 0), (0, padw), (0, 0)))
  dst0 = jnp.pad(dst[:E0].reshape(NS, NH0 * CPH, CH),
                 ((0, 0), (0, padw), (0, 0)), constant_values=N_NODES)
  src1 = jnp.concatenate([src[E0:], jnp.zeros((pad1,), jnp.int32)]
                         ).reshape(NS, NH1 * CPH, CH)
  dst1 = jnp.concatenate([dst[E0:], jnp.full((pad1,), N_NODES, jnp.int32)]
                         ).reshape(NS, NH1 * CPH, CH)
  src_a = jnp.stack([src0, src1])
  dst_a = jnp.stack([dst0, dst1])
  zrows = jnp.zeros((NROWS, 128), jnp.float32)

  r2 = lambda v: v.reshape(1, -1)

  degp = _sc_degree(dst_t, jnp.ones((CH, 128), jnp.float32), zrows)
  rec = _tc_rec(degp)

  s = _tc_stem(x, W_stem, r2(g_stem), r2(b_stem))
  s0, s1 = s, s

  Wp0s = (Wp0_0, Wp0_1)
  Wp1s = (Wp1_0, Wp1_1)
  for i in range(2):
    g01 = jnp.concatenate([pre_g[i, 0], pre_g[i, 1]]).reshape(1, 2 * C)
    b01 = jnp.concatenate([pre_b[i, 0], pre_b[i, 1]]).reshape(1, 2 * C)
    p = _tc_pre(s0, s1, Wp0s[i], Wp1s[i], g01, b01)
    agg1 = _sc_agg(p, src_a, dst_a, zrows)
    q = _tc_post(agg1, rec, opW[i, 0], opW[i, 1], p[:, C:], p[:, :C])
    agg2 = _sc_agg(q, src_a, dst_a, zrows)
    r = _tc_post(agg2, rec, opW[i, 2], opW[i, 3], p[:, C:], q[:, :C])
    s0, s1 = s1, jnp.concatenate([q, r], axis=1)

  return _tc_cls(s1, r2(W_cls[0]), W_cls[1:], r2(b_cls))


# speedup vs baseline: 1.2790x; 1.2790x over previous
import functools

import jax
import jax.numpy as jnp
from jax import lax
from jax.experimental import pallas as pl
from jax.experimental.pallas import tpu as pltpu
from jax.experimental.pallas import tpu_sc as plsc

N_NODES = 10000
N_EDGES = 320000
C = 64

NC = 2
NS = 16
NW = NC * NS
CH = 80
EPT = 10240
NCHUNK = EPT // CH
E_PAD = NW * EPT
NROWS = 10112
RPS = NROWS // NS


NBUF = 4
CPH = 16
NGRP = CPH // NBUF
NH0 = 13
NH1 = 3
NHMAX = max(NH0, NH1)
EPT0 = NH0 * CPH * CH
EPT1 = NH1 * CPH * CH
E0 = NS * EPT0
E1 = NS * EPT1


def _sc_agg(h, src_t, dst_t, zrows):
  mesh = plsc.VectorSubcoreMesh(core_axis_name="c", subcore_axis_name="s")

  @functools.partial(
      pl.kernel,
      out_type=jax.ShapeDtypeStruct((NC, NROWS, 128), jnp.float32),
      mesh=mesh,
      scratch_types=[
          pltpu.VMEM((CPH, CH), jnp.int32),
          pltpu.VMEM((CPH, CH), jnp.int32),
          pltpu.VMEM((NBUF, CH, 128), jnp.float32),
          pltpu.VMEM_SHARED((NROWS, 128), jnp.float32),
      ] + [pltpu.SemaphoreType.DMA] * (2 * NBUF),
  )
  def k(h_hbm, src_hbm, dst_hbm, z_hbm, out_hbm, src_v, dst_v, rows, acc,
        *sems):
    gsem = sems[:NBUF]
    ssem = sems[NBUF:]
    cid = lax.axis_index("c")
    sid = lax.axis_index("s")
    pltpu.sync_copy(z_hbm.at[pl.ds(sid * RPS, RPS)],
                    acc.at[pl.ds(sid * RPS, RPS)])
    plsc.subcore_barrier()

    def round_body(half, carry):
      pltpu.sync_copy(src_hbm.at[cid, sid, pl.ds(half * CPH, CPH)], src_v)
      pltpu.sync_copy(dst_hbm.at[cid, sid, pl.ds(half * CPH, CPH)], dst_v)

      for b in range(NBUF):
        pltpu.async_copy(h_hbm.at[src_v.at[b]], rows.at[b], gsem[b])

      def body(m, carry):
        j0 = m * NBUF
        for b in range(NBUF):
          pltpu.make_async_copy(h_hbm.at[src_v.at[j0 + b]], rows.at[b],
                                gsem[b]).wait()
          pltpu.async_copy(rows.at[b], acc.at[dst_v.at[j0 + b]], ssem[b],
                           add=True)

        @pl.when(m < NGRP - 1)
        def _():
          for b in range(NBUF):
            pltpu.make_async_copy(rows.at[b], acc.at[dst_v.at[j0 + b]],
                                  ssem[b]).wait()
            pltpu.async_copy(h_hbm.at[src_v.at[j0 + NBUF + b]], rows.at[b],
                             gsem[b])

        return carry

      lax.fori_loop(0, NGRP, body, 0)
      for b in range(NBUF):
        pltpu.make_async_copy(rows.at[b], acc.at[dst_v.at[b]], ssem[b]).wait()
      return carry

    @pl.when(cid == 0)
    def _():
      lax.fori_loop(0, NH0, round_body, 0)

    @pl.when(cid == 1)
    def _():
      lax.fori_loop(0, NH1, round_body, 0)

    plsc.subcore_barrier()
    pltpu.sync_copy(acc.at[pl.ds(sid * RPS, RPS)],
                    out_hbm.at[cid, pl.ds(sid * RPS, RPS)])

  return k(h, src_t, dst_t, zrows)


def _sc_degree(dst_t, ones128, zrows):
  mesh = plsc.VectorSubcoreMesh(core_axis_name="c", subcore_axis_name="s")

  @functools.partial(
      pl.kernel,
      out_type=jax.ShapeDtypeStruct((NC, NROWS, 128), jnp.float32),
      mesh=mesh,
      scratch_types=[
          pltpu.VMEM((NCHUNK, CH), jnp.int32),
          pltpu.VMEM((CH, 128), jnp.float32),
          pltpu.VMEM_SHARED((NROWS, 128), jnp.float32),
          pltpu.SemaphoreType.DMA,
      ],
  )
  def k(dst_hbm, ones_hbm, z_hbm, out_hbm, dst_v, ones_v, acc, sem):
    cid = lax.axis_index("c")
    sid = lax.axis_index("s")
    wid = cid * NS + sid
    pltpu.sync_copy(z_hbm.at[pl.ds(sid * RPS, RPS)],
                    acc.at[pl.ds(sid * RPS, RPS)])
    pltpu.sync_copy(dst_hbm.at[wid], dst_v)
    pltpu.sync_copy(ones_hbm, ones_v)
    plsc.subcore_barrier()

    def fire(j, carry):
      pltpu.async_copy(ones_v, acc.at[dst_v.at[j]], sem, add=True)
      return carry

    lax.fori_loop(0, NCHUNK, fire, 0)

    def drain(j, carry):
      pltpu.make_async_copy(ones_v, acc.at[dst_v.at[0]], sem).wait()
      return carry

    lax.fori_loop(0, NCHUNK, drain, 0)
    plsc.subcore_barrier()
    pltpu.sync_copy(acc.at[pl.ds(sid * RPS, RPS)],
                    out_hbm.at[cid, pl.ds(sid * RPS, RPS)])

  return k(dst_t, ones128, zrows)


def _bn_cols(h, g, b):
  m = jnp.mean(h, axis=0, keepdims=True)
  v = jnp.mean((h - m) * (h - m), axis=0, keepdims=True)
  return (h - m) * lax.rsqrt(v + 1e-5) * g + b


def _stem_body(x_ref, w_ref, g_ref, b_ref, o_ref):
  h = jnp.dot(x_ref[...], w_ref[...], preferred_element_type=jnp.float32)
  o_ref[...] = _bn_cols(h, g_ref[...], b_ref[...])


def _tc_stem(x, w, g, b):
  return pl.pallas_call(
      _stem_body,
      out_shape=jax.ShapeDtypeStruct((N_NODES, w.shape[1]), jnp.float32),
  )(x, w, g, b)


def _pre_body(s0_ref, s1_ref, w0_ref, w1_ref, g_ref, b_ref, o_ref):
  h0 = jnp.dot(s0_ref[...], w0_ref[...], preferred_element_type=jnp.float32)
  h1 = jnp.dot(s1_ref[...], w1_ref[...], preferred_element_type=jnp.float32)
  h = jnp.concatenate([h0, h1], axis=1)
  o_ref[...] = jax.nn.relu(_bn_cols(h, g_ref[...], b_ref[...]))


def _tc_pre(s0, s1, w0, w1, g, b):
  return pl.pallas_call(
      _pre_body,
      out_shape=jax.ShapeDtypeStruct((N_NODES, 2 * C), jnp.float32),
  )(s0, s1, w0, w1, g, b)


def _rec_body(d_ref, o_ref):
  deg = d_ref[0, :N_NODES, 0:1] + d_ref[1, :N_NODES, 0:1]
  o_ref[...] = 1.0 / jnp.maximum(deg, 1.0)


def _tc_rec(degp):
  return pl.pallas_call(
      _rec_body,
      out_shape=jax.ShapeDtypeStruct((N_NODES, 1), jnp.float32),
  )(degp)


def _post_body(p_ref, r_ref, wa_ref, wb_ref, ska_ref, skb_ref, o_ref):
  agg = p_ref[0, :N_NODES, :] + p_ref[1, :N_NODES, :]
  rec = r_ref[...]
  a = jnp.dot(agg[:, :C], wa_ref[...], preferred_element_type=jnp.float32)
  bb = jnp.dot(agg[:, C:], wb_ref[...], preferred_element_type=jnp.float32)
  a = jax.nn.relu(a * rec) + ska_ref[...]
  bb = jax.nn.relu(bb * rec) + skb_ref[...]
  o_ref[...] = jnp.concatenate([a, bb], axis=1)


def _tc_post(p, rec, wa, wb, ska, skb):
  return pl.pallas_call(
      _post_body,
      out_shape=jax.ShapeDtypeStruct((N_NODES, 2 * C), jnp.float32),
  )(p, rec, wa, wb, ska, skb)


def _cls_body(s_ref, w0_ref, w1_ref, b_ref, o_ref):
  s = s_ref[...]
  mean = jnp.mean(s, axis=1, keepdims=True)
  o_ref[...] = (jnp.dot(s, w1_ref[...], preferred_element_type=jnp.float32)
                + mean * w0_ref[...] + b_ref[...])


def _tc_cls(s, w0, w1, b):
  return pl.pallas_call(
      _cls_body,
      out_shape=jax.ShapeDtypeStruct((N_NODES, w1.shape[1]), jnp.float32),
  )(s, w0, w1, b)


def kernel(x, edge_index, training, W_stem, g_stem, b_stem, W_pre, g_pre,
           b_pre, Wp0_0, Wp1_0, Wp0_1, Wp1_1, pre_g, pre_b, opW, W_cls,
           b_cls):
  src = edge_index[0]
  dst = edge_index[1]
  pad = E_PAD - N_EDGES
  dst_t = jnp.concatenate([dst, jnp.full((pad,), N_NODES, jnp.int32)]
                          ).reshape(NW, NCHUNK, CH)
  pad1 = E0 + E1 - N_EDGES
  pw0 = (NHMAX - NH0) * CPH
  pw1 = (NHMAX - NH1) * CPH
  src0 = jnp.pad(src[:E0].reshape(NS, NH0 * CPH, CH),
                 ((0, 0), (0, pw0), (0, 0)))
  dst0 = jnp.pad(dst[:E0].reshape(NS, NH0 * CPH, CH),
                 ((0, 0), (0, pw0), (0, 0)), constant_values=N_NODES)
  src1 = jnp.pad(jnp.concatenate([src[E0:], jnp.zeros((pad1,), jnp.int32)]
                                 ).reshape(NS, NH1 * CPH, CH),
                 ((0, 0), (0, pw1), (0, 0)))
  dst1 = jnp.pad(jnp.concatenate([dst[E0:],
                                  jnp.full((pad1,), N_NODES, jnp.int32)]
                                 ).reshape(NS, NH1 * CPH, CH),
                 ((0, 0), (0, pw1), (0, 0)), constant_values=N_NODES)
  src_a = jnp.stack([src0, src1])
  dst_a = jnp.stack([dst0, dst1])
  zrows = jnp.zeros((NROWS, 128), jnp.float32)

  r2 = lambda v: v.reshape(1, -1)

  degp = _sc_degree(dst_t, jnp.ones((CH, 128), jnp.float32), zrows)
  rec = _tc_rec(degp)

  s = _tc_stem(x, W_stem, r2(g_stem), r2(b_stem))
  s0, s1 = s, s

  Wp0s = (Wp0_0, Wp0_1)
  Wp1s = (Wp1_0, Wp1_1)
  for i in range(2):
    g01 = jnp.concatenate([pre_g[i, 0], pre_g[i, 1]]).reshape(1, 2 * C)
    b01 = jnp.concatenate([pre_b[i, 0], pre_b[i, 1]]).reshape(1, 2 * C)
    p = _tc_pre(s0, s1, Wp0s[i], Wp1s[i], g01, b01)
    agg1 = _sc_agg(p, src_a, dst_a, zrows)
    q = _tc_post(agg1, rec, opW[i, 0], opW[i, 1], p[:, C:], p[:, :C])
    agg2 = _sc_agg(q, src_a, dst_a, zrows)
    r = _tc_post(agg2, rec, opW[i, 2], opW[i, 3], p[:, C:], q[:, :C])
    s0, s1 = s1, jnp.concatenate([q, r], axis=1)

  return _tc_cls(s1, r2(W_cls[0]), W_cls[1:], r2(b_cls))
